# baseline (speedup 1.0000x reference)
import functools

import jax
import jax.numpy as jnp
from jax import lax
from jax.experimental import pallas as pl
from jax.experimental.pallas import tpu as pltpu
from jax.experimental.pallas import tpu_sc as plsc

_H = 128
_BN = 1000


def _ln_relu(z, g, b):
    mu = jnp.mean(z, axis=-1, keepdims=True)
    var = jnp.mean((z - mu) ** 2, axis=-1, keepdims=True)
    return jnp.maximum((z - mu) / jnp.sqrt(var + 1e-5) * g + b, 0.0)


def _enc_body(t_ref, x_ref, w_ref, b_ref, h_ref, u_ref):
    h = jnp.dot(x_ref[...], w_ref[...], preferred_element_type=jnp.float32)
    h = h + b_ref[...]
    p = jnp.maximum(h, 0.0) + 1e-7
    ep = jnp.exp(p * t_ref[0, 0])
    h_ref[...] = h
    u_ref[0] = ep.astype(jnp.bfloat16)
    u_ref[1] = (p * ep).astype(jnp.bfloat16)


def _mid_body(t_ref, s_ref, h_ref, w1_ref, b1_ref, g_ref, be_ref, w2_ref,
              b2_ref, lg_ref, lb_ref, h1_ref, r_ref, u_ref):
    out0 = s_ref[1] / (s_ref[0] + 1e-16) + h_ref[...]
    z = jnp.dot(out0, w1_ref[...], preferred_element_type=jnp.float32)
    z = _ln_relu(z + b1_ref[...], g_ref[...], be_ref[...])
    h1 = jnp.dot(z, w2_ref[...], preferred_element_type=jnp.float32)
    h1 = h1 + b2_ref[...]
    r = _ln_relu(h1, lg_ref[...], lb_ref[...])
    p = r + 1e-7
    ep = jnp.exp(p * t_ref[0, 0])
    h1_ref[...] = h1
    r_ref[...] = r
    u_ref[0] = ep.astype(jnp.bfloat16)
    u_ref[1] = (p * ep).astype(jnp.bfloat16)


def _fin_body(s_ref, r_ref, h1_ref, w1_ref, b1_ref, g_ref, be_ref, w2_ref,
              b2_ref, lg_ref, lb_ref, wl_ref, bl_ref, y_ref):
    out1 = s_ref[1] / (s_ref[0] + 1e-16) + r_ref[...]
    z = jnp.dot(out1, w1_ref[...], preferred_element_type=jnp.float32)
    z = _ln_relu(z + b1_ref[...], g_ref[...], be_ref[...])
    hc = jnp.dot(z, w2_ref[...], preferred_element_type=jnp.float32)
    hh = h1_ref[...] + hc + b2_ref[...]
    hf = _ln_relu(hh, lg_ref[...], lb_ref[...])
    y = jnp.dot(hf, wl_ref[...], preferred_element_type=jnp.float32)
    y_ref[...] = y + bl_ref[...]


def _row_spec(r, c):
    return pl.BlockSpec((r, c), lambda i: (jnp.int32(i), jnp.int32(0)))


def _full_spec(r, c):
    return pl.BlockSpec((r, c), lambda i: (jnp.int32(0), jnp.int32(0)))


def _u_spec(n):
    return pl.BlockSpec(
        (2, n, _H), lambda i: (jnp.int32(0), jnp.int32(i), jnp.int32(0)))


def _enc_call(n):
    grid = n // _BN
    return pl.pallas_call(
        _enc_body,
        grid=(grid,),
        in_specs=[_full_spec(1, _H), _row_spec(_BN, _H), _full_spec(_H, _H),
                  _full_spec(1, _H)],
        out_specs=[_row_spec(_BN, _H), _u_spec(_BN)],
        out_shape=[jax.ShapeDtypeStruct((n, _H), jnp.float32),
                   jax.ShapeDtypeStruct((2, n, _H), jnp.bfloat16)],
    )


def _mid_call(n):
    grid = n // _BN
    return pl.pallas_call(
        _mid_body,
        grid=(grid,),
        in_specs=[_full_spec(1, _H), _u_spec(_BN), _row_spec(_BN, _H),
                  _full_spec(_H, 2 * _H), _full_spec(1, 2 * _H),
                  _full_spec(1, 2 * _H), _full_spec(1, 2 * _H),
                  _full_spec(2 * _H, _H), _full_spec(1, _H),
                  _full_spec(1, _H), _full_spec(1, _H)],
        out_specs=[_row_spec(_BN, _H), _row_spec(_BN, _H), _u_spec(_BN)],
        out_shape=[jax.ShapeDtypeStruct((n, _H), jnp.float32),
                   jax.ShapeDtypeStruct((n, _H), jnp.float32),
                   jax.ShapeDtypeStruct((2, n, _H), jnp.bfloat16)],
    )


def _fin_call(n):
    grid = n // _BN
    return pl.pallas_call(
        _fin_body,
        grid=(grid,),
        in_specs=[_u_spec(_BN), _row_spec(_BN, _H), _row_spec(_BN, _H),
                  _full_spec(_H, 2 * _H), _full_spec(1, 2 * _H),
                  _full_spec(1, 2 * _H), _full_spec(1, 2 * _H),
                  _full_spec(2 * _H, _H), _full_spec(1, _H),
                  _full_spec(1, _H), _full_spec(1, _H),
                  _full_spec(_H, _H), _full_spec(1, _H)],
        out_specs=[_row_spec(_BN, _H)],
        out_shape=[jax.ShapeDtypeStruct((n, _H), jnp.float32)],
    )


_NT = 16
_B = 125


def _segsum_call(n, e):
    rt = e // _NT
    nb = rt // _B
    npt = (n // _NT) // 8 * 8
    rem_base = _NT * npt
    rem = n - rem_base
    zr = npt // 6
    nz = npt // zr
    ch = 16
    nch = nb // ch
    mesh = plsc.VectorSubcoreMesh(core_axis_name="c", subcore_axis_name="s")

    @functools.partial(
        pl.kernel,
        out_type=jax.ShapeDtypeStruct((2 * n, _H), jnp.float32),
        mesh=mesh,
        compiler_params=pltpu.CompilerParams(
            use_tc_tiling_on_sc=False, needs_layout_passes=False),
        scratch_types=[
            pltpu.VMEM((ch, _B), jnp.int32),
            pltpu.VMEM((ch, _B), jnp.int32),
            pltpu.VMEM((_B, _H // 2), jnp.int32),
            pltpu.VMEM((_B, _H // 2), jnp.int32),
            pltpu.VMEM((_B, _H), jnp.float32),
            pltpu.VMEM((zr, _H), jnp.float32),
            pltpu.VMEM_SHARED((n, _H), jnp.float32),
            pltpu.SemaphoreType.DMA,
            pltpu.SemaphoreType.DMA,
        ],
    )
    def segsum(table, srcs, dsts, out, srcv, dstv, raw0, raw1, rows, zbuf,
               acc, sem0, sem1):
        c = lax.axis_index("c")
        s = lax.axis_index("s")
        zero16 = jnp.zeros((16,), jnp.float32)

        def zrow(i, carry):
            for j in range(_H // 16):
                zbuf[i, pl.ds(j * 16, 16)] = zero16
            return carry

        lax.fori_loop(jnp.int32(0), jnp.int32(zr), zrow, jnp.int32(0))
        for k in range(nz):
            pltpu.sync_copy(zbuf, acc.at[pl.ds(s * npt + k * zr, zr)])

        @pl.when(s == _NT - 1)
        def _():
            pltpu.sync_copy(zbuf.at[pl.ds(0, rem)],
                            acc.at[pl.ds(rem_base, rem)])

        plsc.subcore_barrier()

        raw = (raw0, raw1)
        sems = (sem0, sem1)

        def widen_from(buf):
            def widen(i, carry):
                for j in range(_H // 32):
                    w = buf[i, pl.ds(j * 16, 16)]
                    sixteen = jnp.full((16,), 16, jnp.int32)
                    mask = jnp.full((16,), -65536, jnp.int32)
                    rows[i, pl.ds(j * 32, 16)] = plsc.bitcast(
                        jnp.left_shift(w, sixteen), jnp.float32)
                    rows[i, pl.ds(j * 32 + 16, 16)] = plsc.bitcast(
                        jnp.bitwise_and(w, mask), jnp.float32)
                return carry
            return widen

        def chunk(k, carry):
            pltpu.sync_copy(srcs.at[c, pl.ds(s * nb + k * ch, ch)], srcv)
            pltpu.sync_copy(dsts.at[pl.ds(s * nb + k * ch, ch)], dstv)
            pend = pltpu.async_copy(table.at[srcv.at[jnp.int32(0)]], raw[0],
                                    sems[0])
            for b in range(ch):
                pend.wait()
                if b + 1 < ch:
                    pend = pltpu.async_copy(table.at[srcv.at[jnp.int32(b + 1)]],
                                            raw[(b + 1) % 2],
                                            sems[(b + 1) % 2])
                lax.fori_loop(jnp.int32(0), jnp.int32(_B),
                              widen_from(raw[b % 2]), jnp.int32(0))
                pltpu.sync_copy(rows, acc.at[dstv.at[jnp.int32(b)]],
                                add=True)
            return carry

        lax.fori_loop(jnp.int32(0), jnp.int32(nch), chunk, jnp.int32(0))
        plsc.subcore_barrier()
        pltpu.sync_copy(acc.at[pl.ds(s * npt, npt)],
                        out.at[pl.ds(c * n + s * npt, npt)])

        @pl.when(s == _NT - 1)
        def _():
            pltpu.sync_copy(acc.at[pl.ds(rem_base, rem)],
                            out.at[pl.ds(c * n + rem_base, rem)])

    return segsum


def kernel(x, edge_index, edge_attr, batch, W_enc, b_enc, t0, c0_W1, c0_b1,
           c0_g, c0_be, c0_W2, c0_b2, ln1_g, ln1_b, t1, c1_W1, c1_b1, c1_g,
           c1_be, c1_W2, c1_b2, ln0_g, ln0_b, W_lin, b_lin):
    n = x.shape[0]
    e = edge_index.shape[1]
    nc = W_lin.shape[1]

    src = edge_index[0].astype(jnp.int32)
    dst = edge_index[1].astype(jnp.int32)
    srcs = jnp.stack([src, src + n]).reshape(2, e // _B, _B)
    dsts = dst.reshape(e // _B, _B)

    xp = jnp.pad(x.astype(jnp.float32), ((0, 0), (0, _H - x.shape[1])))
    wp = jnp.pad(W_enc, ((0, _H - W_enc.shape[0]), (0, 0)))
    wl = jnp.pad(W_lin, ((0, 0), (0, _H - nc)))
    bl = jnp.pad(b_lin, (0, _H - nc))

    t0r = jnp.full((1, _H), t0, jnp.float32)
    t1r = jnp.full((1, _H), t1, jnp.float32)

    segsum = _segsum_call(n, e)

    def unperm(sp):
        sp = sp.reshape(2, n, _H // 32, 2, 16)
        return sp.swapaxes(3, 4).reshape(2, n, _H)

    def pack(u):
        return lax.bitcast_convert_type(
            u.reshape(2 * n, _H // 2, 2), jnp.int32)

    h, u0 = _enc_call(n)(t0r, xp, wp, b_enc.reshape(1, _H))
    s0 = unperm(segsum(pack(u0), srcs, dsts))
    h1, r, u1 = _mid_call(n)(
        t1r, s0, h, c0_W1, c0_b1.reshape(1, 2 * _H), c0_g.reshape(1, 2 * _H),
        c0_be.reshape(1, 2 * _H), c0_W2, c0_b2.reshape(1, _H),
        ln1_g.reshape(1, _H), ln1_b.reshape(1, _H))
    s1 = unperm(segsum(pack(u1), srcs, dsts))
    (y,) = _fin_call(n)(
        s1, r, h1, c1_W1, c1_b1.reshape(1, 2 * _H), c1_g.reshape(1, 2 * _H),
        c1_be.reshape(1, 2 * _H), c1_W2, c1_b2.reshape(1, _H),
        ln0_g.reshape(1, _H), ln0_b.reshape(1, _H), wl, bl.reshape(1, _H))
    return y[:, :nc]


# speedup vs baseline: 2.5869x; 2.5869x over previous
import functools

import jax
import jax.numpy as jnp
from jax import lax
from jax.experimental import pallas as pl
from jax.experimental.pallas import tpu as pltpu
from jax.experimental.pallas import tpu_sc as plsc

_H = 128
_BN = 1000


def _ln_relu(z, g, b):
    mu = jnp.mean(z, axis=-1, keepdims=True)
    var = jnp.mean((z - mu) ** 2, axis=-1, keepdims=True)
    return jnp.maximum((z - mu) / jnp.sqrt(var + 1e-5) * g + b, 0.0)


def _enc_body(t_ref, x_ref, w_ref, b_ref, h_ref, u_ref):
    h = jnp.dot(x_ref[...], w_ref[...], preferred_element_type=jnp.float32)
    h = h + b_ref[...]
    p = jnp.maximum(h, 0.0) + 1e-7
    ep = jnp.exp(p * t_ref[0, 0])
    h_ref[...] = h
    u_ref[0] = ep
    u_ref[1] = p * ep


def _mid_body(t_ref, s_ref, h_ref, w1_ref, b1_ref, g_ref, be_ref, w2_ref,
              b2_ref, lg_ref, lb_ref, h1_ref, r_ref, u_ref):
    out0 = s_ref[1] / (s_ref[0] + 1e-16) + h_ref[...]
    z = jnp.dot(out0, w1_ref[...], preferred_element_type=jnp.float32)
    z = _ln_relu(z + b1_ref[...], g_ref[...], be_ref[...])
    h1 = jnp.dot(z, w2_ref[...], preferred_element_type=jnp.float32)
    h1 = h1 + b2_ref[...]
    r = _ln_relu(h1, lg_ref[...], lb_ref[...])
    p = r + 1e-7
    ep = jnp.exp(p * t_ref[0, 0])
    h1_ref[...] = h1
    r_ref[...] = r
    u_ref[0] = ep
    u_ref[1] = p * ep


def _fin_body(s_ref, r_ref, h1_ref, w1_ref, b1_ref, g_ref, be_ref, w2_ref,
              b2_ref, lg_ref, lb_ref, wl_ref, bl_ref, y_ref):
    out1 = s_ref[1] / (s_ref[0] + 1e-16) + r_ref[...]
    z = jnp.dot(out1, w1_ref[...], preferred_element_type=jnp.float32)
    z = _ln_relu(z + b1_ref[...], g_ref[...], be_ref[...])
    hc = jnp.dot(z, w2_ref[...], preferred_element_type=jnp.float32)
    hh = h1_ref[...] + hc + b2_ref[...]
    hf = _ln_relu(hh, lg_ref[...], lb_ref[...])
    y = jnp.dot(hf, wl_ref[...], preferred_element_type=jnp.float32)
    y_ref[...] = y + bl_ref[...]


def _row_spec(r, c):
    return pl.BlockSpec((r, c), lambda i: (jnp.int32(i), jnp.int32(0)))


def _full_spec(r, c):
    return pl.BlockSpec((r, c), lambda i: (jnp.int32(0), jnp.int32(0)))


def _u_spec(n):
    return pl.BlockSpec(
        (2, n, _H), lambda i: (jnp.int32(0), jnp.int32(i), jnp.int32(0)))


def _enc_call(n):
    grid = n // _BN
    return pl.pallas_call(
        _enc_body,
        grid=(grid,),
        in_specs=[_full_spec(1, _H), _row_spec(_BN, _H), _full_spec(_H, _H),
                  _full_spec(1, _H)],
        out_specs=[_row_spec(_BN, _H), _u_spec(_BN)],
        out_shape=[jax.ShapeDtypeStruct((n, _H), jnp.float32),
                   jax.ShapeDtypeStruct((2, n, _H), jnp.float32)],
    )


def _mid_call(n):
    grid = n // _BN
    return pl.pallas_call(
        _mid_body,
        grid=(grid,),
        in_specs=[_full_spec(1, _H), _u_spec(_BN), _row_spec(_BN, _H),
                  _full_spec(_H, 2 * _H), _full_spec(1, 2 * _H),
                  _full_spec(1, 2 * _H), _full_spec(1, 2 * _H),
                  _full_spec(2 * _H, _H), _full_spec(1, _H),
                  _full_spec(1, _H), _full_spec(1, _H)],
        out_specs=[_row_spec(_BN, _H), _row_spec(_BN, _H), _u_spec(_BN)],
        out_shape=[jax.ShapeDtypeStruct((n, _H), jnp.float32),
                   jax.ShapeDtypeStruct((n, _H), jnp.float32),
                   jax.ShapeDtypeStruct((2, n, _H), jnp.float32)],
    )


def _fin_call(n):
    grid = n // _BN
    return pl.pallas_call(
        _fin_body,
        grid=(grid,),
        in_specs=[_u_spec(_BN), _row_spec(_BN, _H), _row_spec(_BN, _H),
                  _full_spec(_H, 2 * _H), _full_spec(1, 2 * _H),
                  _full_spec(1, 2 * _H), _full_spec(1, 2 * _H),
                  _full_spec(2 * _H, _H), _full_spec(1, _H),
                  _full_spec(1, _H), _full_spec(1, _H),
                  _full_spec(_H, _H), _full_spec(1, _H)],
        out_specs=[_row_spec(_BN, _H)],
        out_shape=[jax.ShapeDtypeStruct((n, _H), jnp.float32)],
    )


_NT = 16
_B = 100


def _segsum_call(n, e):
    rt = e // _NT
    nb = rt // _B
    npt = (n // _NT) // 8 * 8
    rem_base = _NT * npt
    rem = n - rem_base
    zr = 8
    ch = 8
    nch = nb // ch
    nbuf = 3
    mesh = plsc.VectorSubcoreMesh(core_axis_name="c", subcore_axis_name="s")

    @functools.partial(
        pl.kernel,
        out_type=jax.ShapeDtypeStruct((2 * n, _H), jnp.float32),
        mesh=mesh,
        scratch_types=[
            pltpu.VMEM((ch, _B), jnp.int32),
            pltpu.VMEM((ch, _B), jnp.int32),
            pltpu.VMEM((_B, _H), jnp.float32),
            pltpu.VMEM((_B, _H), jnp.float32),
            pltpu.VMEM((_B, _H), jnp.float32),
            pltpu.VMEM((zr, _H), jnp.float32),
            pltpu.VMEM_SHARED((n, _H), jnp.float32),
            pltpu.SemaphoreType.DMA,
            pltpu.SemaphoreType.DMA,
            pltpu.SemaphoreType.DMA,
        ],
    )
    def segsum(table, srcs, dsts, out, srcv, dstv, rows0, rows1, rows2, zbuf,
               acc, sem0, sem1, sem2):
        c = lax.axis_index("c")
        s = lax.axis_index("s")
        zero16 = jnp.zeros((16,), jnp.float32)

        def zrow(i, carry):
            for j in range(_H // 16):
                zbuf[i, pl.ds(j * 16, 16)] = zero16
            return carry

        lax.fori_loop(jnp.int32(0), jnp.int32(zr), zrow, jnp.int32(0))
        for k in range(npt // zr):
            pltpu.sync_copy(zbuf, acc.at[pl.ds(s * npt + k * zr, zr)])

        @pl.when(s == _NT - 1)
        def _():
            for k in range(rem // zr):
                pltpu.sync_copy(zbuf, acc.at[pl.ds(rem_base + k * zr, zr)])

        plsc.subcore_barrier()

        rows = (rows0, rows1, rows2)
        sems = (sem0, sem1, sem2)

        def chunk(k, carry):
            pltpu.sync_copy(srcs.at[c, pl.ds(s * nb + k * ch, ch)], srcv)
            pltpu.sync_copy(dsts.at[pl.ds(s * nb + k * ch, ch)], dstv)
            pend = [None] * ch
            for b in range(min(nbuf - 1, ch)):
                pend[b] = pltpu.async_copy(table.at[srcv.at[jnp.int32(b)]],
                                           rows[b % nbuf], sems[b % nbuf])
            for b in range(ch):
                pend[b].wait()
                nxt = b + nbuf - 1
                if nxt < ch:
                    pend[nxt] = pltpu.async_copy(
                        table.at[srcv.at[jnp.int32(nxt)]],
                        rows[nxt % nbuf], sems[nxt % nbuf])
                pltpu.sync_copy(rows[b % nbuf], acc.at[dstv.at[jnp.int32(b)]],
                                add=True)
            return carry

        lax.fori_loop(jnp.int32(0), jnp.int32(nch), chunk, jnp.int32(0))
        plsc.subcore_barrier()
        pltpu.sync_copy(acc.at[pl.ds(s * npt, npt)],
                        out.at[pl.ds(c * n + s * npt, npt)])

        @pl.when(s == _NT - 1)
        def _():
            pltpu.sync_copy(acc.at[pl.ds(rem_base, rem)],
                            out.at[pl.ds(c * n + rem_base, rem)])

    return segsum


def kernel(x, edge_index, edge_attr, batch, W_enc, b_enc, t0, c0_W1, c0_b1,
           c0_g, c0_be, c0_W2, c0_b2, ln1_g, ln1_b, t1, c1_W1, c1_b1, c1_g,
           c1_be, c1_W2, c1_b2, ln0_g, ln0_b, W_lin, b_lin):
    n = x.shape[0]
    e = edge_index.shape[1]
    nc = W_lin.shape[1]

    src = edge_index[0].astype(jnp.int32)
    dst = edge_index[1].astype(jnp.int32)
    srcs = jnp.stack([src, src + n]).reshape(2, e // _B, _B)
    dsts = dst.reshape(e // _B, _B)

    xp = jnp.pad(x.astype(jnp.float32), ((0, 0), (0, _H - x.shape[1])))
    wp = jnp.pad(W_enc, ((0, _H - W_enc.shape[0]), (0, 0)))
    wl = jnp.pad(W_lin, ((0, 0), (0, _H - nc)))
    bl = jnp.pad(b_lin, (0, _H - nc))

    t0r = jnp.full((1, _H), t0, jnp.float32)
    t1r = jnp.full((1, _H), t1, jnp.float32)

    segsum = _segsum_call(n, e)

    h, u0 = _enc_call(n)(t0r, xp, wp, b_enc.reshape(1, _H))
    s0 = segsum(u0.reshape(2 * n, _H), srcs, dsts).reshape(2, n, _H)
    h1, r, u1 = _mid_call(n)(
        t1r, s0, h, c0_W1, c0_b1.reshape(1, 2 * _H), c0_g.reshape(1, 2 * _H),
        c0_be.reshape(1, 2 * _H), c0_W2, c0_b2.reshape(1, _H),
        ln1_g.reshape(1, _H), ln1_b.reshape(1, _H))
    s1 = segsum(u1.reshape(2 * n, _H), srcs, dsts).reshape(2, n, _H)
    (y,) = _fin_call(n)(
        s1, r, h1, c1_W1, c1_b1.reshape(1, 2 * _H), c1_g.reshape(1, 2 * _H),
        c1_be.reshape(1, 2 * _H), c1_W2, c1_b2.reshape(1, _H),
        ln0_g.reshape(1, _H), ln0_b.reshape(1, _H), wl, bl.reshape(1, _H))
    return y[:, :nc]


# speedup vs baseline: 2.9371x; 1.1354x over previous
import functools

import jax
import jax.numpy as jnp
from jax import lax
from jax.experimental import pallas as pl
from jax.experimental.pallas import tpu as pltpu
from jax.experimental.pallas import tpu_sc as plsc

_H = 128
_BN = 1000


def _ln_relu(z, g, b):
    mu = jnp.mean(z, axis=-1, keepdims=True)
    var = jnp.mean((z - mu) ** 2, axis=-1, keepdims=True)
    return jnp.maximum((z - mu) / jnp.sqrt(var + 1e-5) * g + b, 0.0)


def _enc_body(t_ref, x_ref, w_ref, b_ref, h_ref, u_ref):
    h = jnp.dot(x_ref[...], w_ref[...], preferred_element_type=jnp.float32)
    h = h + b_ref[...]
    p = jnp.maximum(h, 0.0) + 1e-7
    ep = jnp.exp(p * t_ref[0, 0])
    h_ref[...] = h
    u_ref[0] = ep
    u_ref[1] = p * ep


def _mid_body(t_ref, s_ref, h_ref, w1_ref, b1_ref, g_ref, be_ref, w2_ref,
              b2_ref, lg_ref, lb_ref, h1_ref, r_ref, u_ref):
    out0 = s_ref[1] / (s_ref[0] + 1e-16) + h_ref[...]
    z = jnp.dot(out0, w1_ref[...], preferred_element_type=jnp.float32)
    z = _ln_relu(z + b1_ref[...], g_ref[...], be_ref[...])
    h1 = jnp.dot(z, w2_ref[...], preferred_element_type=jnp.float32)
    h1 = h1 + b2_ref[...]
    r = _ln_relu(h1, lg_ref[...], lb_ref[...])
    p = r + 1e-7
    ep = jnp.exp(p * t_ref[0, 0])
    h1_ref[...] = h1
    r_ref[...] = r
    u_ref[0] = ep
    u_ref[1] = p * ep


def _fin_body(s_ref, r_ref, h1_ref, w1_ref, b1_ref, g_ref, be_ref, w2_ref,
              b2_ref, lg_ref, lb_ref, wl_ref, bl_ref, y_ref):
    out1 = s_ref[1] / (s_ref[0] + 1e-16) + r_ref[...]
    z = jnp.dot(out1, w1_ref[...], preferred_element_type=jnp.float32)
    z = _ln_relu(z + b1_ref[...], g_ref[...], be_ref[...])
    hc = jnp.dot(z, w2_ref[...], preferred_element_type=jnp.float32)
    hh = h1_ref[...] + hc + b2_ref[...]
    hf = _ln_relu(hh, lg_ref[...], lb_ref[...])
    y = jnp.dot(hf, wl_ref[...], preferred_element_type=jnp.float32)
    y_ref[...] = y + bl_ref[...]


def _row_spec(r, c):
    return pl.BlockSpec((r, c), lambda i: (jnp.int32(i), jnp.int32(0)))


def _full_spec(r, c):
    return pl.BlockSpec((r, c), lambda i: (jnp.int32(0), jnp.int32(0)))


def _u_spec(n):
    return pl.BlockSpec(
        (2, n, _H), lambda i: (jnp.int32(0), jnp.int32(i), jnp.int32(0)))


def _enc_call(n):
    grid = n // _BN
    return pl.pallas_call(
        _enc_body,
        grid=(grid,),
        in_specs=[_full_spec(1, _H), _row_spec(_BN, _H), _full_spec(_H, _H),
                  _full_spec(1, _H)],
        out_specs=[_row_spec(_BN, _H), _u_spec(_BN)],
        out_shape=[jax.ShapeDtypeStruct((n, _H), jnp.float32),
                   jax.ShapeDtypeStruct((2, n, _H), jnp.float32)],
    )


def _mid_call(n):
    grid = n // _BN
    return pl.pallas_call(
        _mid_body,
        grid=(grid,),
        in_specs=[_full_spec(1, _H), _u_spec(_BN), _row_spec(_BN, _H),
                  _full_spec(_H, 2 * _H), _full_spec(1, 2 * _H),
                  _full_spec(1, 2 * _H), _full_spec(1, 2 * _H),
                  _full_spec(2 * _H, _H), _full_spec(1, _H),
                  _full_spec(1, _H), _full_spec(1, _H)],
        out_specs=[_row_spec(_BN, _H), _row_spec(_BN, _H), _u_spec(_BN)],
        out_shape=[jax.ShapeDtypeStruct((n, _H), jnp.float32),
                   jax.ShapeDtypeStruct((n, _H), jnp.float32),
                   jax.ShapeDtypeStruct((2, n, _H), jnp.float32)],
    )


def _fin_call(n):
    grid = n // _BN
    return pl.pallas_call(
        _fin_body,
        grid=(grid,),
        in_specs=[_u_spec(_BN), _row_spec(_BN, _H), _row_spec(_BN, _H),
                  _full_spec(_H, 2 * _H), _full_spec(1, 2 * _H),
                  _full_spec(1, 2 * _H), _full_spec(1, 2 * _H),
                  _full_spec(2 * _H, _H), _full_spec(1, _H),
                  _full_spec(1, _H), _full_spec(1, _H),
                  _full_spec(_H, _H), _full_spec(1, _H)],
        out_specs=[_row_spec(_BN, _H)],
        out_shape=[jax.ShapeDtypeStruct((n, _H), jnp.float32)],
    )


_NT = 16
_B = 100


def _segsum_call(n, e):
    rt = e // _NT
    nb = rt // _B
    npt = (n // _NT) // 8 * 8
    rem_base = _NT * npt
    rem = n - rem_base
    zr = 8
    ch = 16
    nch = nb // ch
    nbuf = 3
    mesh = plsc.VectorSubcoreMesh(core_axis_name="c", subcore_axis_name="s")

    @functools.partial(
        pl.kernel,
        out_type=jax.ShapeDtypeStruct((2 * n, _H), jnp.float32),
        mesh=mesh,
        scratch_types=[
            pltpu.VMEM((ch, _B), jnp.int32),
            pltpu.VMEM((ch, _B), jnp.int32),
            pltpu.VMEM((_B, _H), jnp.float32),
            pltpu.VMEM((_B, _H), jnp.float32),
            pltpu.VMEM((_B, _H), jnp.float32),
            pltpu.VMEM((zr, _H), jnp.float32),
            pltpu.VMEM_SHARED((n, _H), jnp.float32),
            pltpu.SemaphoreType.DMA,
            pltpu.SemaphoreType.DMA,
            pltpu.SemaphoreType.DMA,
        ],
    )
    def segsum(table, srcs, dsts, out, srcv, dstv, rows0, rows1, rows2, zbuf,
               acc, sem0, sem1, sem2):
        c = lax.axis_index("c")
        s = lax.axis_index("s")
        zero16 = jnp.zeros((16,), jnp.float32)

        def zrow(i, carry):
            for j in range(_H // 16):
                zbuf[i, pl.ds(j * 16, 16)] = zero16
            return carry

        lax.fori_loop(jnp.int32(0), jnp.int32(zr), zrow, jnp.int32(0))
        for k in range(npt // zr):
            pltpu.sync_copy(zbuf, acc.at[pl.ds(s * npt + k * zr, zr)])

        @pl.when(s == _NT - 1)
        def _():
            for k in range(rem // zr):
                pltpu.sync_copy(zbuf, acc.at[pl.ds(rem_base + k * zr, zr)])

        plsc.subcore_barrier()

        rows = (rows0, rows1, rows2)
        sems = (sem0, sem1, sem2)

        def chunk(k, carry):
            pltpu.sync_copy(srcs.at[c, pl.ds(s * nb + k * ch, ch)], srcv)
            pltpu.sync_copy(dsts.at[pl.ds(s * nb + k * ch, ch)], dstv)
            pend = [None] * ch
            for b in range(min(nbuf - 1, ch)):
                pend[b] = pltpu.async_copy(table.at[srcv.at[jnp.int32(b)]],
                                           rows[b % nbuf], sems[b % nbuf])
            for b in range(ch):
                pend[b].wait()
                nxt = b + nbuf - 1
                if nxt < ch:
                    pend[nxt] = pltpu.async_copy(
                        table.at[srcv.at[jnp.int32(nxt)]],
                        rows[nxt % nbuf], sems[nxt % nbuf])
                pltpu.sync_copy(rows[b % nbuf], acc.at[dstv.at[jnp.int32(b)]],
                                add=True)
            return carry

        lax.fori_loop(jnp.int32(0), jnp.int32(nch), chunk, jnp.int32(0))
        plsc.subcore_barrier()
        pltpu.sync_copy(acc.at[pl.ds(s * npt, npt)],
                        out.at[pl.ds(c * n + s * npt, npt)])

        @pl.when(s == _NT - 1)
        def _():
            pltpu.sync_copy(acc.at[pl.ds(rem_base, rem)],
                            out.at[pl.ds(c * n + rem_base, rem)])

    return segsum


def kernel(x, edge_index, edge_attr, batch, W_enc, b_enc, t0, c0_W1, c0_b1,
           c0_g, c0_be, c0_W2, c0_b2, ln1_g, ln1_b, t1, c1_W1, c1_b1, c1_g,
           c1_be, c1_W2, c1_b2, ln0_g, ln0_b, W_lin, b_lin):
    n = x.shape[0]
    e = edge_index.shape[1]
    nc = W_lin.shape[1]

    src = edge_index[0].astype(jnp.int32)
    dst = edge_index[1].astype(jnp.int32)
    srcs = jnp.stack([src, src + n]).reshape(2, e // _B, _B)
    dsts = dst.reshape(e // _B, _B)

    xp = jnp.pad(x.astype(jnp.float32), ((0, 0), (0, _H - x.shape[1])))
    wp = jnp.pad(W_enc, ((0, _H - W_enc.shape[0]), (0, 0)))
    wl = jnp.pad(W_lin, ((0, 0), (0, _H - nc)))
    bl = jnp.pad(b_lin, (0, _H - nc))

    t0r = jnp.full((1, _H), t0, jnp.float32)
    t1r = jnp.full((1, _H), t1, jnp.float32)

    segsum = _segsum_call(n, e)

    h, u0 = _enc_call(n)(t0r, xp, wp, b_enc.reshape(1, _H))
    s0 = segsum(u0.reshape(2 * n, _H), srcs, dsts).reshape(2, n, _H)
    h1, r, u1 = _mid_call(n)(
        t1r, s0, h, c0_W1, c0_b1.reshape(1, 2 * _H), c0_g.reshape(1, 2 * _H),
        c0_be.reshape(1, 2 * _H), c0_W2, c0_b2.reshape(1, _H),
        ln1_g.reshape(1, _H), ln1_b.reshape(1, _H))
    s1 = segsum(u1.reshape(2 * n, _H), srcs, dsts).reshape(2, n, _H)
    (y,) = _fin_call(n)(
        s1, r, h1, c1_W1, c1_b1.reshape(1, 2 * _H), c1_g.reshape(1, 2 * _H),
        c1_be.reshape(1, 2 * _H), c1_W2, c1_b2.reshape(1, _H),
        ln0_g.reshape(1, _H), ln0_b.reshape(1, _H), wl, bl.reshape(1, _H))
    return y[:, :nc]


# speedup vs baseline: 3.0640x; 1.0432x over previous
import functools

import jax
import jax.numpy as jnp
from jax import lax
from jax.experimental import pallas as pl
from jax.experimental.pallas import tpu as pltpu
from jax.experimental.pallas import tpu_sc as plsc

_H = 128
_BN = 1000


def _ln_relu(z, g, b):
    mu = jnp.mean(z, axis=-1, keepdims=True)
    var = jnp.mean((z - mu) ** 2, axis=-1, keepdims=True)
    return jnp.maximum((z - mu) / jnp.sqrt(var + 1e-5) * g + b, 0.0)


def _enc_body(t_ref, x_ref, w_ref, b_ref, h_ref, u_ref):
    h = jnp.dot(x_ref[...], w_ref[...], preferred_element_type=jnp.float32)
    h = h + b_ref[...]
    p = jnp.maximum(h, 0.0) + 1e-7
    ep = jnp.exp(p * t_ref[0, 0])
    h_ref[...] = h
    u_ref[0] = ep
    u_ref[1] = p * ep


def _mid_body(t_ref, s_ref, h_ref, w1_ref, b1_ref, g_ref, be_ref, w2_ref,
              b2_ref, lg_ref, lb_ref, h1_ref, r_ref, u_ref):
    out0 = s_ref[1] / (s_ref[0] + 1e-16) + h_ref[...]
    z = jnp.dot(out0, w1_ref[...], preferred_element_type=jnp.float32)
    z = _ln_relu(z + b1_ref[...], g_ref[...], be_ref[...])
    h1 = jnp.dot(z, w2_ref[...], preferred_element_type=jnp.float32)
    h1 = h1 + b2_ref[...]
    r = _ln_relu(h1, lg_ref[...], lb_ref[...])
    p = r + 1e-7
    ep = jnp.exp(p * t_ref[0, 0])
    h1_ref[...] = h1
    r_ref[...] = r
    u_ref[0] = ep
    u_ref[1] = p * ep


def _fin_body(s_ref, r_ref, h1_ref, w1_ref, b1_ref, g_ref, be_ref, w2_ref,
              b2_ref, lg_ref, lb_ref, wl_ref, bl_ref, y_ref):
    out1 = s_ref[1] / (s_ref[0] + 1e-16) + r_ref[...]
    z = jnp.dot(out1, w1_ref[...], preferred_element_type=jnp.float32)
    z = _ln_relu(z + b1_ref[...], g_ref[...], be_ref[...])
    hc = jnp.dot(z, w2_ref[...], preferred_element_type=jnp.float32)
    hh = h1_ref[...] + hc + b2_ref[...]
    hf = _ln_relu(hh, lg_ref[...], lb_ref[...])
    y = jnp.dot(hf, wl_ref[...], preferred_element_type=jnp.float32)
    y_ref[...] = y + bl_ref[...]


def _row_spec(r, c):
    return pl.BlockSpec((r, c), lambda i: (jnp.int32(i), jnp.int32(0)))


def _full_spec(r, c):
    return pl.BlockSpec((r, c), lambda i: (jnp.int32(0), jnp.int32(0)))


def _u_spec(n):
    return pl.BlockSpec(
        (2, n, _H), lambda i: (jnp.int32(0), jnp.int32(i), jnp.int32(0)))


def _enc_call(n):
    grid = n // _BN
    return pl.pallas_call(
        _enc_body,
        grid=(grid,),
        in_specs=[_full_spec(1, _H), _row_spec(_BN, _H), _full_spec(_H, _H),
                  _full_spec(1, _H)],
        out_specs=[_row_spec(_BN, _H), _u_spec(_BN)],
        out_shape=[jax.ShapeDtypeStruct((n, _H), jnp.float32),
                   jax.ShapeDtypeStruct((2, n, _H), jnp.float32)],
    )


def _mid_call(n):
    grid = n // _BN
    return pl.pallas_call(
        _mid_body,
        grid=(grid,),
        in_specs=[_full_spec(1, _H), _u_spec(_BN), _row_spec(_BN, _H),
                  _full_spec(_H, 2 * _H), _full_spec(1, 2 * _H),
                  _full_spec(1, 2 * _H), _full_spec(1, 2 * _H),
                  _full_spec(2 * _H, _H), _full_spec(1, _H),
                  _full_spec(1, _H), _full_spec(1, _H)],
        out_specs=[_row_spec(_BN, _H), _row_spec(_BN, _H), _u_spec(_BN)],
        out_shape=[jax.ShapeDtypeStruct((n, _H), jnp.float32),
                   jax.ShapeDtypeStruct((n, _H), jnp.float32),
                   jax.ShapeDtypeStruct((2, n, _H), jnp.float32)],
    )


def _fin_call(n):
    grid = n // _BN
    return pl.pallas_call(
        _fin_body,
        grid=(grid,),
        in_specs=[_u_spec(_BN), _row_spec(_BN, _H), _row_spec(_BN, _H),
                  _full_spec(_H, 2 * _H), _full_spec(1, 2 * _H),
                  _full_spec(1, 2 * _H), _full_spec(1, 2 * _H),
                  _full_spec(2 * _H, _H), _full_spec(1, _H),
                  _full_spec(1, _H), _full_spec(1, _H),
                  _full_spec(_H, _H), _full_spec(1, _H)],
        out_specs=[_row_spec(_BN, _H)],
        out_shape=[jax.ShapeDtypeStruct((n, _H), jnp.float32)],
    )


_NT = 16
_B = 100


def _segsum_call(n, e):
    rt = e // _NT
    nb = rt // _B
    npt = (n // _NT) // 8 * 8
    rem_base = _NT * npt
    rem = n - rem_base
    zr = 8
    ch = 40
    assert nb % ch == 0 and (ch % 8 == 0)
    nch = nb // ch
    nbuf = 3
    mesh = plsc.VectorSubcoreMesh(core_axis_name="c", subcore_axis_name="s")

    @functools.partial(
        pl.kernel,
        out_type=jax.ShapeDtypeStruct((2 * n, _H), jnp.float32),
        mesh=mesh,
        scratch_types=[
            pltpu.VMEM((ch, _B), jnp.int32),
            pltpu.VMEM((ch, _B), jnp.int32),
            pltpu.VMEM((_B, _H), jnp.float32),
            pltpu.VMEM((_B, _H), jnp.float32),
            pltpu.VMEM((_B, _H), jnp.float32),
            pltpu.VMEM_SHARED((n, _H), jnp.float32),
            pltpu.SemaphoreType.DMA,
            pltpu.SemaphoreType.DMA,
            pltpu.SemaphoreType.DMA,
        ],
    )
    def segsum(table, srcs, dsts, out, srcv, dstv, rows0, rows1, rows2,
               acc, sem0, sem1, sem2):
        c = lax.axis_index("c")
        s = lax.axis_index("s")
        zero16 = jnp.zeros((16,), jnp.float32)

        def zrow(i, carry):
            for j in range(_H // 16):
                rows0[i, pl.ds(j * 16, 16)] = zero16
            return carry

        lax.fori_loop(jnp.int32(0), jnp.int32(zr), zrow, jnp.int32(0))
        zsrc = rows0.at[pl.ds(0, zr)]
        for k in range(npt // zr):
            pltpu.sync_copy(zsrc, acc.at[pl.ds(s * npt + k * zr, zr)])

        @pl.when(s == _NT - 1)
        def _():
            for k in range(rem // zr):
                pltpu.sync_copy(zsrc, acc.at[pl.ds(rem_base + k * zr, zr)])

        plsc.subcore_barrier()

        rows = (rows0, rows1, rows2)
        sems = (sem0, sem1, sem2)

        def chunk(k, carry):
            pltpu.sync_copy(srcs.at[c, pl.ds(s * nb + k * ch, ch)], srcv)
            pltpu.sync_copy(dsts.at[pl.ds(s * nb + k * ch, ch)], dstv)
            pend = [None] * ch
            for b in range(min(nbuf - 1, ch)):
                pend[b] = pltpu.async_copy(table.at[srcv.at[jnp.int32(b)]],
                                           rows[b % nbuf], sems[b % nbuf])
            for b in range(ch):
                pend[b].wait()
                nxt = b + nbuf - 1
                if nxt < ch:
                    pend[nxt] = pltpu.async_copy(
                        table.at[srcv.at[jnp.int32(nxt)]],
                        rows[nxt % nbuf], sems[nxt % nbuf])
                pltpu.sync_copy(rows[b % nbuf], acc.at[dstv.at[jnp.int32(b)]],
                                add=True)
            return carry

        lax.fori_loop(jnp.int32(0), jnp.int32(nch), chunk, jnp.int32(0))
        plsc.subcore_barrier()
        pltpu.sync_copy(acc.at[pl.ds(s * npt, npt)],
                        out.at[pl.ds(c * n + s * npt, npt)])

        @pl.when(s == _NT - 1)
        def _():
            pltpu.sync_copy(acc.at[pl.ds(rem_base, rem)],
                            out.at[pl.ds(c * n + rem_base, rem)])

    return segsum


def kernel(x, edge_index, edge_attr, batch, W_enc, b_enc, t0, c0_W1, c0_b1,
           c0_g, c0_be, c0_W2, c0_b2, ln1_g, ln1_b, t1, c1_W1, c1_b1, c1_g,
           c1_be, c1_W2, c1_b2, ln0_g, ln0_b, W_lin, b_lin):
    n = x.shape[0]
    e = edge_index.shape[1]
    nc = W_lin.shape[1]

    src = edge_index[0].astype(jnp.int32)
    dst = edge_index[1].astype(jnp.int32)
    srcs = jnp.stack([src, src + n]).reshape(2, e // _B, _B)
    dsts = dst.reshape(e // _B, _B)

    xp = jnp.pad(x.astype(jnp.float32), ((0, 0), (0, _H - x.shape[1])))
    wp = jnp.pad(W_enc, ((0, _H - W_enc.shape[0]), (0, 0)))
    wl = jnp.pad(W_lin, ((0, 0), (0, _H - nc)))
    bl = jnp.pad(b_lin, (0, _H - nc))

    t0r = jnp.full((1, _H), t0, jnp.float32)
    t1r = jnp.full((1, _H), t1, jnp.float32)

    segsum = _segsum_call(n, e)

    h, u0 = _enc_call(n)(t0r, xp, wp, b_enc.reshape(1, _H))
    s0 = segsum(u0.reshape(2 * n, _H), srcs, dsts).reshape(2, n, _H)
    h1, r, u1 = _mid_call(n)(
        t1r, s0, h, c0_W1, c0_b1.reshape(1, 2 * _H), c0_g.reshape(1, 2 * _H),
        c0_be.reshape(1, 2 * _H), c0_W2, c0_b2.reshape(1, _H),
        ln1_g.reshape(1, _H), ln1_b.reshape(1, _H))
    s1 = segsum(u1.reshape(2 * n, _H), srcs, dsts).reshape(2, n, _H)
    (y,) = _fin_call(n)(
        s1, r, h1, c1_W1, c1_b1.reshape(1, 2 * _H), c1_g.reshape(1, 2 * _H),
        c1_be.reshape(1, 2 * _H), c1_W2, c1_b2.reshape(1, _H),
        ln0_g.reshape(1, _H), ln0_b.reshape(1, _H), wl, bl.reshape(1, _H))
    return y[:, :nc]


# speedup vs baseline: 3.1148x; 1.0166x over previous
import functools

import jax
import jax.numpy as jnp
from jax import lax
from jax.experimental import pallas as pl
from jax.experimental.pallas import tpu as pltpu
from jax.experimental.pallas import tpu_sc as plsc

_H = 128
_BN = 2000


def _ln_relu(z, g, b):
    mu = jnp.mean(z, axis=-1, keepdims=True)
    var = jnp.mean((z - mu) ** 2, axis=-1, keepdims=True)
    return jnp.maximum((z - mu) / jnp.sqrt(var + 1e-5) * g + b, 0.0)


def _enc_body(t_ref, x_ref, w_ref, b_ref, h_ref, u_ref):
    h = jnp.dot(x_ref[...], w_ref[...], preferred_element_type=jnp.float32)
    h = h + b_ref[...]
    p = jnp.maximum(h, 0.0) + 1e-7
    ep = jnp.exp(p * t_ref[0, 0])
    h_ref[...] = h
    u_ref[0] = ep
    u_ref[1] = p * ep


def _mid_body(t_ref, s_ref, h_ref, w1_ref, b1_ref, g_ref, be_ref, w2_ref,
              b2_ref, lg_ref, lb_ref, h1_ref, r_ref, u_ref):
    out0 = s_ref[1] / (s_ref[0] + 1e-16) + h_ref[...]
    z = jnp.dot(out0, w1_ref[...], preferred_element_type=jnp.float32)
    z = _ln_relu(z + b1_ref[...], g_ref[...], be_ref[...])
    h1 = jnp.dot(z, w2_ref[...], preferred_element_type=jnp.float32)
    h1 = h1 + b2_ref[...]
    r = _ln_relu(h1, lg_ref[...], lb_ref[...])
    p = r + 1e-7
    ep = jnp.exp(p * t_ref[0, 0])
    h1_ref[...] = h1
    r_ref[...] = r
    u_ref[0] = ep
    u_ref[1] = p * ep


def _fin_body(s_ref, r_ref, h1_ref, w1_ref, b1_ref, g_ref, be_ref, w2_ref,
              b2_ref, lg_ref, lb_ref, wl_ref, bl_ref, y_ref):
    out1 = s_ref[1] / (s_ref[0] + 1e-16) + r_ref[...]
    z = jnp.dot(out1, w1_ref[...], preferred_element_type=jnp.float32)
    z = _ln_relu(z + b1_ref[...], g_ref[...], be_ref[...])
    hc = jnp.dot(z, w2_ref[...], preferred_element_type=jnp.float32)
    hh = h1_ref[...] + hc + b2_ref[...]
    hf = _ln_relu(hh, lg_ref[...], lb_ref[...])
    y = jnp.dot(hf, wl_ref[...], preferred_element_type=jnp.float32)
    y_ref[...] = y + bl_ref[...]


def _row_spec(r, c):
    return pl.BlockSpec((r, c), lambda i: (jnp.int32(i), jnp.int32(0)))


def _full_spec(r, c):
    return pl.BlockSpec((r, c), lambda i: (jnp.int32(0), jnp.int32(0)))


def _u_spec(n):
    return pl.BlockSpec(
        (2, n, _H), lambda i: (jnp.int32(0), jnp.int32(i), jnp.int32(0)))


def _enc_call(n):
    grid = n // _BN
    return pl.pallas_call(
        _enc_body,
        grid=(grid,),
        in_specs=[_full_spec(1, _H), _row_spec(_BN, _H), _full_spec(_H, _H),
                  _full_spec(1, _H)],
        out_specs=[_row_spec(_BN, _H), _u_spec(_BN)],
        out_shape=[jax.ShapeDtypeStruct((n, _H), jnp.float32),
                   jax.ShapeDtypeStruct((2, n, _H), jnp.float32)],
    )


def _mid_call(n):
    grid = n // _BN
    return pl.pallas_call(
        _mid_body,
        grid=(grid,),
        in_specs=[_full_spec(1, _H), _u_spec(_BN), _row_spec(_BN, _H),
                  _full_spec(_H, 2 * _H), _full_spec(1, 2 * _H),
                  _full_spec(1, 2 * _H), _full_spec(1, 2 * _H),
                  _full_spec(2 * _H, _H), _full_spec(1, _H),
                  _full_spec(1, _H), _full_spec(1, _H)],
        out_specs=[_row_spec(_BN, _H), _row_spec(_BN, _H), _u_spec(_BN)],
        out_shape=[jax.ShapeDtypeStruct((n, _H), jnp.float32),
                   jax.ShapeDtypeStruct((n, _H), jnp.float32),
                   jax.ShapeDtypeStruct((2, n, _H), jnp.float32)],
    )


def _fin_call(n):
    grid = n // _BN
    return pl.pallas_call(
        _fin_body,
        grid=(grid,),
        in_specs=[_u_spec(_BN), _row_spec(_BN, _H), _row_spec(_BN, _H),
                  _full_spec(_H, 2 * _H), _full_spec(1, 2 * _H),
                  _full_spec(1, 2 * _H), _full_spec(1, 2 * _H),
                  _full_spec(2 * _H, _H), _full_spec(1, _H),
                  _full_spec(1, _H), _full_spec(1, _H),
                  _full_spec(_H, _H), _full_spec(1, _H)],
        out_specs=[_row_spec(_BN, _H)],
        out_shape=[jax.ShapeDtypeStruct((n, _H), jnp.float32)],
    )


_NT = 16
_B = 100


def _segsum_call(n, e):
    rt = e // _NT
    nb = rt // _B
    npt = (n // _NT) // 8 * 8
    rem_base = _NT * npt
    rem = n - rem_base
    zr = 8
    ch = 40
    assert nb % ch == 0 and (ch % 8 == 0)
    nch = nb // ch
    nbuf = 3
    mesh = plsc.VectorSubcoreMesh(core_axis_name="c", subcore_axis_name="s")

    @functools.partial(
        pl.kernel,
        out_type=jax.ShapeDtypeStruct((2 * n, _H), jnp.float32),
        mesh=mesh,
        scratch_types=[
            pltpu.VMEM((ch, _B), jnp.int32),
            pltpu.VMEM((ch, _B), jnp.int32),
            pltpu.VMEM((_B, _H), jnp.float32),
            pltpu.VMEM((_B, _H), jnp.float32),
            pltpu.VMEM((_B, _H), jnp.float32),
            pltpu.VMEM_SHARED((n, _H), jnp.float32),
            pltpu.SemaphoreType.DMA,
            pltpu.SemaphoreType.DMA,
            pltpu.SemaphoreType.DMA,
        ],
    )
    def segsum(table, srcs, dsts, out, srcv, dstv, rows0, rows1, rows2,
               acc, sem0, sem1, sem2):
        c = lax.axis_index("c")
        s = lax.axis_index("s")
        zero16 = jnp.zeros((16,), jnp.float32)

        def zrow(i, carry):
            for j in range(_H // 16):
                rows0[i, pl.ds(j * 16, 16)] = zero16
            return carry

        lax.fori_loop(jnp.int32(0), jnp.int32(zr), zrow, jnp.int32(0))
        zsrc = rows0.at[pl.ds(0, zr)]
        for k in range(npt // zr):
            pltpu.sync_copy(zsrc, acc.at[pl.ds(s * npt + k * zr, zr)])

        @pl.when(s == _NT - 1)
        def _():
            for k in range(rem // zr):
                pltpu.sync_copy(zsrc, acc.at[pl.ds(rem_base + k * zr, zr)])

        plsc.subcore_barrier()

        rows = (rows0, rows1, rows2)
        sems = (sem0, sem1, sem2)

        def chunk(k, carry):
            pltpu.sync_copy(srcs.at[c, pl.ds(s * nb + k * ch, ch)], srcv)
            pltpu.sync_copy(dsts.at[pl.ds(s * nb + k * ch, ch)], dstv)
            pend = [None] * ch
            for b in range(min(nbuf - 1, ch)):
                pend[b] = pltpu.async_copy(table.at[srcv.at[jnp.int32(b)]],
                                           rows[b % nbuf], sems[b % nbuf])
            for b in range(ch):
                pend[b].wait()
                nxt = b + nbuf - 1
                if nxt < ch:
                    pend[nxt] = pltpu.async_copy(
                        table.at[srcv.at[jnp.int32(nxt)]],
                        rows[nxt % nbuf], sems[nxt % nbuf])
                pltpu.sync_copy(rows[b % nbuf], acc.at[dstv.at[jnp.int32(b)]],
                                add=True)
            return carry

        lax.fori_loop(jnp.int32(0), jnp.int32(nch), chunk, jnp.int32(0))
        plsc.subcore_barrier()
        pltpu.sync_copy(acc.at[pl.ds(s * npt, npt)],
                        out.at[pl.ds(c * n + s * npt, npt)])

        @pl.when(s == _NT - 1)
        def _():
            pltpu.sync_copy(acc.at[pl.ds(rem_base, rem)],
                            out.at[pl.ds(c * n + rem_base, rem)])

    return segsum


def kernel(x, edge_index, edge_attr, batch, W_enc, b_enc, t0, c0_W1, c0_b1,
           c0_g, c0_be, c0_W2, c0_b2, ln1_g, ln1_b, t1, c1_W1, c1_b1, c1_g,
           c1_be, c1_W2, c1_b2, ln0_g, ln0_b, W_lin, b_lin):
    n = x.shape[0]
    e = edge_index.shape[1]
    nc = W_lin.shape[1]

    src = edge_index[0].astype(jnp.int32)
    dst = edge_index[1].astype(jnp.int32)
    srcs = jnp.stack([src, src + n]).reshape(2, e // _B, _B)
    dsts = dst.reshape(e // _B, _B)

    xp = jnp.pad(x.astype(jnp.float32), ((0, 0), (0, _H - x.shape[1])))
    wp = jnp.pad(W_enc, ((0, _H - W_enc.shape[0]), (0, 0)))
    wl = jnp.pad(W_lin, ((0, 0), (0, _H - nc)))
    bl = jnp.pad(b_lin, (0, _H - nc))

    t0r = jnp.full((1, _H), t0, jnp.float32)
    t1r = jnp.full((1, _H), t1, jnp.float32)

    segsum = _segsum_call(n, e)

    h, u0 = _enc_call(n)(t0r, xp, wp, b_enc.reshape(1, _H))
    s0 = segsum(u0.reshape(2 * n, _H), srcs, dsts).reshape(2, n, _H)
    h1, r, u1 = _mid_call(n)(
        t1r, s0, h, c0_W1, c0_b1.reshape(1, 2 * _H), c0_g.reshape(1, 2 * _H),
        c0_be.reshape(1, 2 * _H), c0_W2, c0_b2.reshape(1, _H),
        ln1_g.reshape(1, _H), ln1_b.reshape(1, _H))
    s1 = segsum(u1.reshape(2 * n, _H), srcs, dsts).reshape(2, n, _H)
    (y,) = _fin_call(n)(
        s1, r, h1, c1_W1, c1_b1.reshape(1, 2 * _H), c1_g.reshape(1, 2 * _H),
        c1_be.reshape(1, 2 * _H), c1_W2, c1_b2.reshape(1, _H),
        ln0_g.reshape(1, _H), ln0_b.reshape(1, _H), wl, bl.reshape(1, _H))
    return y[:, :nc]


# speedup vs baseline: 3.1218x; 1.0023x over previous
import functools

import jax
import jax.numpy as jnp
from jax import lax
from jax.experimental import pallas as pl
from jax.experimental.pallas import tpu as pltpu
from jax.experimental.pallas import tpu_sc as plsc

_H = 128
_BN = 5000


def _ln_relu(z, g, b):
    mu = jnp.mean(z, axis=-1, keepdims=True)
    var = jnp.mean((z - mu) ** 2, axis=-1, keepdims=True)
    return jnp.maximum((z - mu) / jnp.sqrt(var + 1e-5) * g + b, 0.0)


def _enc_body(t_ref, x_ref, w_ref, b_ref, h_ref, u_ref):
    h = jnp.dot(x_ref[...], w_ref[...], preferred_element_type=jnp.float32)
    h = h + b_ref[...]
    p = jnp.maximum(h, 0.0) + 1e-7
    ep = jnp.exp(p * t_ref[0, 0])
    h_ref[...] = h
    u_ref[0] = ep
    u_ref[1] = p * ep


def _mid_body(t_ref, s_ref, h_ref, w1_ref, b1_ref, g_ref, be_ref, w2_ref,
              b2_ref, lg_ref, lb_ref, h1_ref, r_ref, u_ref):
    out0 = s_ref[1] / (s_ref[0] + 1e-16) + h_ref[...]
    z = jnp.dot(out0, w1_ref[...], preferred_element_type=jnp.float32)
    z = _ln_relu(z + b1_ref[...], g_ref[...], be_ref[...])
    h1 = jnp.dot(z, w2_ref[...], preferred_element_type=jnp.float32)
    h1 = h1 + b2_ref[...]
    r = _ln_relu(h1, lg_ref[...], lb_ref[...])
    p = r + 1e-7
    ep = jnp.exp(p * t_ref[0, 0])
    h1_ref[...] = h1
    r_ref[...] = r
    u_ref[0] = ep
    u_ref[1] = p * ep


def _fin_body(s_ref, r_ref, h1_ref, w1_ref, b1_ref, g_ref, be_ref, w2_ref,
              b2_ref, lg_ref, lb_ref, wl_ref, bl_ref, y_ref):
    out1 = s_ref[1] / (s_ref[0] + 1e-16) + r_ref[...]
    z = jnp.dot(out1, w1_ref[...], preferred_element_type=jnp.float32)
    z = _ln_relu(z + b1_ref[...], g_ref[...], be_ref[...])
    hc = jnp.dot(z, w2_ref[...], preferred_element_type=jnp.float32)
    hh = h1_ref[...] + hc + b2_ref[...]
    hf = _ln_relu(hh, lg_ref[...], lb_ref[...])
    y = jnp.dot(hf, wl_ref[...], preferred_element_type=jnp.float32)
    y_ref[...] = y + bl_ref[...]


def _row_spec(r, c):
    return pl.BlockSpec((r, c), lambda i: (jnp.int32(i), jnp.int32(0)))


def _full_spec(r, c):
    return pl.BlockSpec((r, c), lambda i: (jnp.int32(0), jnp.int32(0)))


def _u_spec(n):
    return pl.BlockSpec(
        (2, n, _H), lambda i: (jnp.int32(0), jnp.int32(i), jnp.int32(0)))


def _enc_call(n):
    grid = n // _BN
    return pl.pallas_call(
        _enc_body,
        grid=(grid,),
        in_specs=[_full_spec(1, _H), _row_spec(_BN, _H), _full_spec(_H, _H),
                  _full_spec(1, _H)],
        out_specs=[_row_spec(_BN, _H), _u_spec(_BN)],
        out_shape=[jax.ShapeDtypeStruct((n, _H), jnp.float32),
                   jax.ShapeDtypeStruct((2, n, _H), jnp.float32)],
    )


def _mid_call(n):
    grid = n // _BN
    return pl.pallas_call(
        _mid_body,
        grid=(grid,),
        in_specs=[_full_spec(1, _H), _u_spec(_BN), _row_spec(_BN, _H),
                  _full_spec(_H, 2 * _H), _full_spec(1, 2 * _H),
                  _full_spec(1, 2 * _H), _full_spec(1, 2 * _H),
                  _full_spec(2 * _H, _H), _full_spec(1, _H),
                  _full_spec(1, _H), _full_spec(1, _H)],
        out_specs=[_row_spec(_BN, _H), _row_spec(_BN, _H), _u_spec(_BN)],
        out_shape=[jax.ShapeDtypeStruct((n, _H), jnp.float32),
                   jax.ShapeDtypeStruct((n, _H), jnp.float32),
                   jax.ShapeDtypeStruct((2, n, _H), jnp.float32)],
    )


def _fin_call(n):
    grid = n // _BN
    return pl.pallas_call(
        _fin_body,
        grid=(grid,),
        in_specs=[_u_spec(_BN), _row_spec(_BN, _H), _row_spec(_BN, _H),
                  _full_spec(_H, 2 * _H), _full_spec(1, 2 * _H),
                  _full_spec(1, 2 * _H), _full_spec(1, 2 * _H),
                  _full_spec(2 * _H, _H), _full_spec(1, _H),
                  _full_spec(1, _H), _full_spec(1, _H),
                  _full_spec(_H, _H), _full_spec(1, _H)],
        out_specs=[_row_spec(_BN, _H)],
        out_shape=[jax.ShapeDtypeStruct((n, _H), jnp.float32)],
    )


_NT = 16
_B = 100


def _segsum_call(n, e):
    rt = e // _NT
    nb = rt // _B
    npt = (n // _NT) // 8 * 8
    rem_base = _NT * npt
    rem = n - rem_base
    zr = 8
    ch = 40
    assert nb % ch == 0 and (ch % 8 == 0)
    nch = nb // ch
    nbuf = 3
    mesh = plsc.VectorSubcoreMesh(core_axis_name="c", subcore_axis_name="s")

    @functools.partial(
        pl.kernel,
        out_type=jax.ShapeDtypeStruct((2 * n, _H), jnp.float32),
        mesh=mesh,
        scratch_types=[
            pltpu.VMEM((ch, _B), jnp.int32),
            pltpu.VMEM((ch, _B), jnp.int32),
            pltpu.VMEM((_B, _H), jnp.float32),
            pltpu.VMEM((_B, _H), jnp.float32),
            pltpu.VMEM((_B, _H), jnp.float32),
            pltpu.VMEM_SHARED((n, _H), jnp.float32),
            pltpu.SemaphoreType.DMA,
            pltpu.SemaphoreType.DMA,
            pltpu.SemaphoreType.DMA,
        ],
    )
    def segsum(table, srcs, dsts, out, srcv, dstv, rows0, rows1, rows2,
               acc, sem0, sem1, sem2):
        c = lax.axis_index("c")
        s = lax.axis_index("s")
        zero16 = jnp.zeros((16,), jnp.float32)

        def zrow(i, carry):
            for j in range(_H // 16):
                rows0[i, pl.ds(j * 16, 16)] = zero16
            return carry

        lax.fori_loop(jnp.int32(0), jnp.int32(zr), zrow, jnp.int32(0))
        zsrc = rows0.at[pl.ds(0, zr)]
        for k in range(npt // zr):
            pltpu.sync_copy(zsrc, acc.at[pl.ds(s * npt + k * zr, zr)])

        @pl.when(s == _NT - 1)
        def _():
            for k in range(rem // zr):
                pltpu.sync_copy(zsrc, acc.at[pl.ds(rem_base + k * zr, zr)])

        plsc.subcore_barrier()

        rows = (rows0, rows1, rows2)
        sems = (sem0, sem1, sem2)

        def chunk(k, carry):
            pltpu.sync_copy(srcs.at[c, pl.ds(s * nb + k * ch, ch)], srcv)
            pltpu.sync_copy(dsts.at[pl.ds(s * nb + k * ch, ch)], dstv)
            pend = [None] * ch
            for b in range(min(nbuf - 1, ch)):
                pend[b] = pltpu.async_copy(table.at[srcv.at[jnp.int32(b)]],
                                           rows[b % nbuf], sems[b % nbuf])
            for b in range(ch):
                pend[b].wait()
                nxt = b + nbuf - 1
                if nxt < ch:
                    pend[nxt] = pltpu.async_copy(
                        table.at[srcv.at[jnp.int32(nxt)]],
                        rows[nxt % nbuf], sems[nxt % nbuf])
                pltpu.sync_copy(rows[b % nbuf], acc.at[dstv.at[jnp.int32(b)]],
                                add=True)
            return carry

        lax.fori_loop(jnp.int32(0), jnp.int32(nch), chunk, jnp.int32(0))
        plsc.subcore_barrier()
        pltpu.sync_copy(acc.at[pl.ds(s * npt, npt)],
                        out.at[pl.ds(c * n + s * npt, npt)])

        @pl.when(s == _NT - 1)
        def _():
            pltpu.sync_copy(acc.at[pl.ds(rem_base, rem)],
                            out.at[pl.ds(c * n + rem_base, rem)])

    return segsum


def kernel(x, edge_index, edge_attr, batch, W_enc, b_enc, t0, c0_W1, c0_b1,
           c0_g, c0_be, c0_W2, c0_b2, ln1_g, ln1_b, t1, c1_W1, c1_b1, c1_g,
           c1_be, c1_W2, c1_b2, ln0_g, ln0_b, W_lin, b_lin):
    n = x.shape[0]
    e = edge_index.shape[1]
    nc = W_lin.shape[1]

    src = edge_index[0].astype(jnp.int32)
    dst = edge_index[1].astype(jnp.int32)
    srcs = jnp.stack([src, src + n]).reshape(2, e // _B, _B)
    dsts = dst.reshape(e // _B, _B)

    xp = jnp.pad(x.astype(jnp.float32), ((0, 0), (0, _H - x.shape[1])))
    wp = jnp.pad(W_enc, ((0, _H - W_enc.shape[0]), (0, 0)))
    wl = jnp.pad(W_lin, ((0, 0), (0, _H - nc)))
    bl = jnp.pad(b_lin, (0, _H - nc))

    t0r = jnp.full((1, _H), t0, jnp.float32)
    t1r = jnp.full((1, _H), t1, jnp.float32)

    segsum = _segsum_call(n, e)

    h, u0 = _enc_call(n)(t0r, xp, wp, b_enc.reshape(1, _H))
    s0 = segsum(u0.reshape(2 * n, _H), srcs, dsts).reshape(2, n, _H)
    h1, r, u1 = _mid_call(n)(
        t1r, s0, h, c0_W1, c0_b1.reshape(1, 2 * _H), c0_g.reshape(1, 2 * _H),
        c0_be.reshape(1, 2 * _H), c0_W2, c0_b2.reshape(1, _H),
        ln1_g.reshape(1, _H), ln1_b.reshape(1, _H))
    s1 = segsum(u1.reshape(2 * n, _H), srcs, dsts).reshape(2, n, _H)
    (y,) = _fin_call(n)(
        s1, r, h1, c1_W1, c1_b1.reshape(1, 2 * _H), c1_g.reshape(1, 2 * _H),
        c1_be.reshape(1, 2 * _H), c1_W2, c1_b2.reshape(1, _H),
        ln0_g.reshape(1, _H), ln0_b.reshape(1, _H), wl, bl.reshape(1, _H))
    return y[:, :nc]
